# Initial kernel scaffold; baseline (speedup 1.0000x reference)
import jax
import jax.numpy as jnp
from jax.experimental import pallas as pl


def kernel(x, edge_index, W0, b0, W1, b1, W2, b2, g0, beta0, g1, beta1):
    raise NotImplementedError("write your pallas kernel here")


# speedup vs baseline: 5.3672x; 5.3672x over previous
import functools

import jax
import jax.numpy as jnp
from jax import lax
from jax.experimental import pallas as pl
from jax.experimental.pallas import tpu as pltpu
from jax.experimental.pallas import tpu_sc as plsc

N = 10000
E = 320000
EPS = 1e-5

_info = plsc.get_sparse_core_info()
NC = _info.num_cores
NS = _info.num_subcores
NW = NC * NS
LANES = _info.num_lanes

EPT = E // NW
CHUNK = 80
NCHUNK = EPT // CHUNK
SLAB = 624
ZCH = 16

_MESH = dict(mesh=plsc.VectorSubcoreMesh(core_axis_name="c",
                                         subcore_axis_name="s"),
             compiler_params=pltpu.CompilerParams(needs_layout_passes=False,
                                                  use_tc_tiling_on_sc=False))


@functools.partial(
    pl.kernel,
    out_type=jax.ShapeDtypeStruct((NW, 2 * N), jnp.float32),
    scratch_types=[
        pltpu.VMEM((EPT,), jnp.int32),
        pltpu.VMEM((EPT,), jnp.int32),
        pltpu.VMEM((2 * N,), jnp.float32),
    ],
    **_MESH,
)
def _deg_kernel(src, dst, out, sbuf, dbuf, hist):
    wid = lax.axis_index("s") * NC + lax.axis_index("c")
    base = wid * EPT
    pltpu.sync_copy(src.at[pl.ds(base, EPT)], sbuf)
    pltpu.sync_copy(dst.at[pl.ds(base, EPT)], dbuf)
    zeros16 = jnp.zeros((LANES,), jnp.float32)

    def zbody(i, c):
        hist[pl.ds(i * LANES, LANES)] = zeros16
        return c

    lax.fori_loop(0, (2 * N) // LANES, zbody, 0)
    ones16 = jnp.ones((LANES,), jnp.float32)
    off = jnp.full((LANES,), N, jnp.int32)

    def body(i, c):
        si = sbuf[pl.ds(i * LANES, LANES)]
        plsc.addupdate_scatter(hist, [si], ones16)
        di = dbuf[pl.ds(i * LANES, LANES)]
        plsc.addupdate_scatter(hist, [di + off], ones16)
        return c

    lax.fori_loop(0, EPT // LANES, body, 0)
    pltpu.sync_copy(hist, out.at[wid])


def _make_seg(D):
    @functools.partial(
        pl.kernel,
        out_type=jax.ShapeDtypeStruct((NC, N, D), jnp.float32),
        scratch_types=[
            pltpu.VMEM((CHUNK,), jnp.int32),
            pltpu.VMEM((CHUNK,), jnp.int32),
            pltpu.VMEM((CHUNK, D), jnp.float32),
            pltpu.VMEM((ZCH, D), jnp.float32),
            pltpu.VMEM_SHARED((N, D), jnp.float32),
            pltpu.SemaphoreType.DMA,
        ],
        **_MESH,
    )
    def _seg(y, src, dst, out, sbuf, dbuf, rows, zbuf, acc, sem):
        cid = lax.axis_index("c")
        sid = lax.axis_index("s")
        wid = sid * NC + cid
        zeros16 = jnp.zeros((LANES,), jnp.float32)
        dl = D // LANES

        def zb(i, c):
            zbuf[i // dl, pl.ds((i % dl) * LANES, LANES)] = zeros16
            return c

        lax.fori_loop(0, ZCH * dl, zb, 0)
        rbase = sid * SLAB

        def zacc(i, c):
            pltpu.sync_copy(zbuf, acc.at[pl.ds(rbase + i * ZCH, ZCH)])
            return c

        lax.fori_loop(0, SLAB // ZCH, zacc, 0)

        @pl.when(sid == NS - 1)
        def _():
            lax.fori_loop(SLAB // ZCH, (N - (NS - 1) * SLAB) // ZCH, zacc, 0)

        plsc.subcore_barrier()
        ebase = wid * EPT

        def chunk(k, c):
            pltpu.sync_copy(src.at[pl.ds(ebase + k * CHUNK, CHUNK)], sbuf)
            pltpu.sync_copy(dst.at[pl.ds(ebase + k * CHUNK, CHUNK)], dbuf)
            pltpu.async_copy(y.at[sbuf], rows, sem).wait()
            pltpu.sync_copy(rows, acc.at[dbuf], add=True)
            return c

        lax.fori_loop(0, NCHUNK, chunk, 0)
        plsc.subcore_barrier()

        @pl.when(sid != NS - 1)
        def _():
            pltpu.sync_copy(acc.at[pl.ds(rbase, SLAB)],
                            out.at[cid, pl.ds(rbase, SLAB)])

        @pl.when(sid == NS - 1)
        def _():
            pltpu.sync_copy(acc.at[pl.ds(rbase, N - (NS - 1) * SLAB)],
                            out.at[cid, pl.ds(rbase, N - (NS - 1) * SLAB)])

    return _seg


_seg128 = _make_seg(128)
_seg64 = _make_seg(64)


RB = 1000
GRID = N // RB


def _prep_body(degp_ref, dis_ref):
    deg = jnp.sum(degp_ref[...], axis=0)
    dis_ref[...] = jnp.where(deg > 0.0,
                             lax.rsqrt(jnp.maximum(deg, 1.0)), 0.0)


_prep = pl.pallas_call(
    _prep_body,
    out_shape=jax.ShapeDtypeStruct((2 * N,), jnp.float32),
)


def _lin0_body(x_ref, diso_ref, w_ref, y_ref):
    y_ref[...] = jnp.dot(x_ref[...] * diso_ref[...], w_ref[...],
                         preferred_element_type=jnp.float32)


_lin0 = pl.pallas_call(
    _lin0_body,
    grid=(GRID,),
    in_specs=[
        pl.BlockSpec((RB, 128), lambda i: (i, 0)),
        pl.BlockSpec((RB, 1), lambda i: (i, 0)),
        pl.BlockSpec((128, 128), lambda i: (0, 0)),
    ],
    out_specs=pl.BlockSpec((RB, 128), lambda i: (i, 0)),
    out_shape=jax.ShapeDtypeStruct((N, 128), jnp.float32),
)


def _stats_body(z0_ref, z1_ref, disi_ref, b_ref, s_ref):
    h = ((z0_ref[...] + z1_ref[...]) * disi_ref[...]
         + b_ref[...][None, :])
    blk = jnp.stack([jnp.sum(h, axis=0), jnp.sum(h * h, axis=0)])

    @pl.when(pl.program_id(0) == 0)
    def _():
        s_ref[...] = blk

    @pl.when(pl.program_id(0) != 0)
    def _():
        s_ref[...] = s_ref[...] + blk


_stats = pl.pallas_call(
    _stats_body,
    grid=(GRID,),
    in_specs=[
        pl.BlockSpec((RB, 128), lambda i: (i, 0)),
        pl.BlockSpec((RB, 128), lambda i: (i, 0)),
        pl.BlockSpec((RB, 1), lambda i: (i, 0)),
        pl.BlockSpec((128,), lambda i: (0,)),
    ],
    out_specs=pl.BlockSpec((2, 128), lambda i: (0, 0)),
    out_shape=jax.ShapeDtypeStruct((2, 128), jnp.float32),
)


def _lin_body(z0_ref, z1_ref, disi_ref, b_ref, s_ref, g_ref, beta_ref,
              diso_ref, w_ref, y_ref):
    h = ((z0_ref[...] + z1_ref[...]) * disi_ref[...]
         + b_ref[...][None, :])
    mu = s_ref[0, :] / N
    var = jnp.maximum(s_ref[1, :] / N - mu * mu, 0.0)
    a = (h - mu[None, :]) * lax.rsqrt(var + EPS)[None, :]
    a = jnp.maximum(a * g_ref[...][None, :] + beta_ref[...][None, :], 0.0)
    y_ref[...] = jnp.dot(a * diso_ref[...], w_ref[...],
                         preferred_element_type=jnp.float32)


def _make_lin(dout):
    return pl.pallas_call(
        _lin_body,
        grid=(GRID,),
        in_specs=[
            pl.BlockSpec((RB, 128), lambda i: (i, 0)),
            pl.BlockSpec((RB, 128), lambda i: (i, 0)),
            pl.BlockSpec((RB, 1), lambda i: (i, 0)),
            pl.BlockSpec((128,), lambda i: (0,)),
            pl.BlockSpec((2, 128), lambda i: (0, 0)),
            pl.BlockSpec((128,), lambda i: (0,)),
            pl.BlockSpec((128,), lambda i: (0,)),
            pl.BlockSpec((RB, 1), lambda i: (i, 0)),
            pl.BlockSpec((128, dout), lambda i: (0, 0)),
        ],
        out_specs=pl.BlockSpec((RB, dout), lambda i: (i, 0)),
        out_shape=jax.ShapeDtypeStruct((N, dout), jnp.float32),
    )


_lin1 = _make_lin(128)
_lin2 = _make_lin(64)


def _final_body(z0_ref, z1_ref, disi_ref, b_ref, o_ref):
    o_ref[...] = ((z0_ref[...] + z1_ref[...]) * disi_ref[...]
                  + b_ref[...][None, :])


_final = pl.pallas_call(
    _final_body,
    grid=(GRID,),
    in_specs=[
        pl.BlockSpec((RB, 64), lambda i: (i, 0)),
        pl.BlockSpec((RB, 64), lambda i: (i, 0)),
        pl.BlockSpec((RB, 1), lambda i: (i, 0)),
        pl.BlockSpec((64,), lambda i: (0,)),
    ],
    out_specs=pl.BlockSpec((RB, 64), lambda i: (i, 0)),
    out_shape=jax.ShapeDtypeStruct((N, 64), jnp.float32),
)


def kernel(x, edge_index, W0, b0, W1, b1, W2, b2, g0, beta0, g1, beta1):
    src, dst = edge_index[0], edge_index[1]
    degp = _deg_kernel(src, dst)
    dis = _prep(degp)
    diso, disi = dis[:N, None], dis[N:, None]
    y0 = _lin0(x, diso, W0)
    z0 = _seg128(y0, src, dst)
    s0 = _stats(z0[0], z0[1], disi, b0)
    y1 = _lin1(z0[0], z0[1], disi, b0, s0, g0, beta0, diso, W1)
    z1 = _seg128(y1, src, dst)
    s1 = _stats(z1[0], z1[1], disi, b1)
    W2p = jnp.pad(W2, ((0, 0), (0, 24)))
    b2p = jnp.pad(b2, (0, 24))
    y2 = _lin2(z1[0], z1[1], disi, b1, s1, g1, beta1, diso, W2p)
    z2 = _seg64(y2, src, dst)
    out = _final(z2[0], z2[1], disi, b2p)
    return out[:, :40]


# speedup vs baseline: 11.5459x; 2.1512x over previous
import functools

import jax
import jax.numpy as jnp
from jax import lax
from jax.experimental import pallas as pl
from jax.experimental.pallas import tpu as pltpu
from jax.experimental.pallas import tpu_sc as plsc

N = 10000
E = 320000
EPS = 1e-5

_info = plsc.get_sparse_core_info()
NC = _info.num_cores
NS = _info.num_subcores
NW = NC * NS
LANES = _info.num_lanes

EPT = E // NW
CHUNK = 80
NCHUNK = EPT // CHUNK
SLAB = 624
ZCH = 16

_MESH = dict(mesh=plsc.VectorSubcoreMesh(core_axis_name="c",
                                         subcore_axis_name="s"),
             compiler_params=pltpu.CompilerParams(needs_layout_passes=False,
                                                  use_tc_tiling_on_sc=False))


@functools.partial(
    pl.kernel,
    out_type=jax.ShapeDtypeStruct((NW, 2 * N), jnp.float32),
    scratch_types=[
        pltpu.VMEM((EPT,), jnp.int32),
        pltpu.VMEM((EPT,), jnp.int32),
        pltpu.VMEM((2 * N,), jnp.float32),
    ],
    **_MESH,
)
def _deg_kernel(src, dst, out, sbuf, dbuf, hist):
    wid = lax.axis_index("s") * NC + lax.axis_index("c")
    base = wid * EPT
    pltpu.sync_copy(src.at[pl.ds(base, EPT)], sbuf)
    pltpu.sync_copy(dst.at[pl.ds(base, EPT)], dbuf)
    zeros16 = jnp.zeros((LANES,), jnp.float32)

    def zbody(i, c):
        hist[pl.ds(i * LANES, LANES)] = zeros16
        return c

    lax.fori_loop(0, (2 * N) // LANES, zbody, 0)
    ones16 = jnp.ones((LANES,), jnp.float32)
    off = jnp.full((LANES,), N, jnp.int32)

    def body(i, c):
        si = sbuf[pl.ds(i * LANES, LANES)]
        plsc.addupdate_scatter(hist, [si], ones16)
        di = dbuf[pl.ds(i * LANES, LANES)]
        plsc.addupdate_scatter(hist, [di + off], ones16)
        return c

    lax.fori_loop(0, EPT // LANES, body, 0)
    pltpu.sync_copy(hist, out.at[wid])


def _make_seg(D):
    @functools.partial(
        pl.kernel,
        out_type=jax.ShapeDtypeStruct((NC, N, D), jnp.float32),
        scratch_types=[
            pltpu.VMEM((NCHUNK, CHUNK), jnp.int32),
            pltpu.VMEM((NCHUNK, CHUNK), jnp.int32),
            pltpu.VMEM((CHUNK, D), jnp.float32),
            pltpu.VMEM((CHUNK, D), jnp.float32),
            pltpu.VMEM((ZCH, D), jnp.float32),
            pltpu.VMEM_SHARED((N, D), jnp.float32),
            pltpu.SemaphoreType.DMA,
            pltpu.SemaphoreType.DMA,
            pltpu.SemaphoreType.DMA,
            pltpu.SemaphoreType.DMA,
        ],
        **_MESH,
    )
    def _seg(y, src3, dst3, out, sidx, didx, rows0, rows1, zbuf, acc,
             gsem0, gsem1, isem0, isem1):
        cid = lax.axis_index("c")
        sid = lax.axis_index("s")
        wid = sid * NC + cid
        icp0 = pltpu.async_copy(src3.at[wid], sidx, isem0)
        icp1 = pltpu.async_copy(dst3.at[wid], didx, isem1)
        zeros16 = jnp.zeros((LANES,), jnp.float32)
        dl = D // LANES

        def zb(i, c):
            zbuf[i // dl, pl.ds((i % dl) * LANES, LANES)] = zeros16
            return c

        lax.fori_loop(0, ZCH * dl, zb, 0)
        rbase = sid * SLAB

        def zacc(i, c):
            pltpu.sync_copy(zbuf, acc.at[pl.ds(rbase + i * ZCH, ZCH)])
            return c

        lax.fori_loop(0, SLAB // ZCH, zacc, 0)

        @pl.when(sid == NS - 1)
        def _():
            lax.fori_loop(SLAB // ZCH, (N - (NS - 1) * SLAB) // ZCH, zacc, 0)

        icp0.wait()
        icp1.wait()
        plsc.subcore_barrier()

        pltpu.async_copy(y.at[sidx.at[0]], rows0, gsem0)

        def chunk(k, c):
            @pl.when(k % 2 == 0)
            def _():
                @pl.when(k + 1 < NCHUNK)
                def _():
                    pltpu.async_copy(y.at[sidx.at[k + 1]], rows1, gsem1)

                pltpu.make_async_copy(y.at[sidx.at[k]], rows0, gsem0).wait()
                pltpu.sync_copy(rows0, acc.at[didx.at[k]], add=True)

            @pl.when(k % 2 == 1)
            def _():
                @pl.when(k + 1 < NCHUNK)
                def _():
                    pltpu.async_copy(y.at[sidx.at[k + 1]], rows0, gsem0)

                pltpu.make_async_copy(y.at[sidx.at[k]], rows1, gsem1).wait()
                pltpu.sync_copy(rows1, acc.at[didx.at[k]], add=True)

            return c

        lax.fori_loop(0, NCHUNK, chunk, 0)
        plsc.subcore_barrier()

        @pl.when(sid != NS - 1)
        def _():
            pltpu.sync_copy(acc.at[pl.ds(rbase, SLAB)],
                            out.at[cid, pl.ds(rbase, SLAB)])

        @pl.when(sid == NS - 1)
        def _():
            pltpu.sync_copy(acc.at[pl.ds(rbase, N - (NS - 1) * SLAB)],
                            out.at[cid, pl.ds(rbase, N - (NS - 1) * SLAB)])

    return _seg


_seg128 = _make_seg(128)
_seg64 = _make_seg(64)


RB = 1000
GRID = N // RB


def _prep_body(degp_ref, dis_ref):
    deg = jnp.sum(degp_ref[...], axis=0)
    dis_ref[...] = jnp.where(deg > 0.0,
                             lax.rsqrt(jnp.maximum(deg, 1.0)), 0.0)


_prep = pl.pallas_call(
    _prep_body,
    out_shape=jax.ShapeDtypeStruct((2 * N,), jnp.float32),
)


def _lin0_body(x_ref, diso_ref, w_ref, y_ref):
    y_ref[...] = jnp.dot(x_ref[...] * diso_ref[...], w_ref[...],
                         preferred_element_type=jnp.float32)


_lin0 = pl.pallas_call(
    _lin0_body,
    grid=(GRID,),
    in_specs=[
        pl.BlockSpec((RB, 128), lambda i: (i, 0)),
        pl.BlockSpec((RB, 1), lambda i: (i, 0)),
        pl.BlockSpec((128, 128), lambda i: (0, 0)),
    ],
    out_specs=pl.BlockSpec((RB, 128), lambda i: (i, 0)),
    out_shape=jax.ShapeDtypeStruct((N, 128), jnp.float32),
)


def _stats_body(z0_ref, z1_ref, disi_ref, b_ref, s_ref):
    h = ((z0_ref[...] + z1_ref[...]) * disi_ref[...]
         + b_ref[...][None, :])
    blk = jnp.stack([jnp.sum(h, axis=0), jnp.sum(h * h, axis=0)])

    @pl.when(pl.program_id(0) == 0)
    def _():
        s_ref[...] = blk

    @pl.when(pl.program_id(0) != 0)
    def _():
        s_ref[...] = s_ref[...] + blk


_stats = pl.pallas_call(
    _stats_body,
    grid=(GRID,),
    in_specs=[
        pl.BlockSpec((RB, 128), lambda i: (i, 0)),
        pl.BlockSpec((RB, 128), lambda i: (i, 0)),
        pl.BlockSpec((RB, 1), lambda i: (i, 0)),
        pl.BlockSpec((128,), lambda i: (0,)),
    ],
    out_specs=pl.BlockSpec((2, 128), lambda i: (0, 0)),
    out_shape=jax.ShapeDtypeStruct((2, 128), jnp.float32),
)


def _lin_body(z0_ref, z1_ref, disi_ref, b_ref, s_ref, g_ref, beta_ref,
              diso_ref, w_ref, y_ref):
    h = ((z0_ref[...] + z1_ref[...]) * disi_ref[...]
         + b_ref[...][None, :])
    mu = s_ref[0, :] / N
    var = jnp.maximum(s_ref[1, :] / N - mu * mu, 0.0)
    a = (h - mu[None, :]) * lax.rsqrt(var + EPS)[None, :]
    a = jnp.maximum(a * g_ref[...][None, :] + beta_ref[...][None, :], 0.0)
    y_ref[...] = jnp.dot(a * diso_ref[...], w_ref[...],
                         preferred_element_type=jnp.float32)


def _make_lin(dout):
    return pl.pallas_call(
        _lin_body,
        grid=(GRID,),
        in_specs=[
            pl.BlockSpec((RB, 128), lambda i: (i, 0)),
            pl.BlockSpec((RB, 128), lambda i: (i, 0)),
            pl.BlockSpec((RB, 1), lambda i: (i, 0)),
            pl.BlockSpec((128,), lambda i: (0,)),
            pl.BlockSpec((2, 128), lambda i: (0, 0)),
            pl.BlockSpec((128,), lambda i: (0,)),
            pl.BlockSpec((128,), lambda i: (0,)),
            pl.BlockSpec((RB, 1), lambda i: (i, 0)),
            pl.BlockSpec((128, dout), lambda i: (0, 0)),
        ],
        out_specs=pl.BlockSpec((RB, dout), lambda i: (i, 0)),
        out_shape=jax.ShapeDtypeStruct((N, dout), jnp.float32),
    )


_lin1 = _make_lin(128)
_lin2 = _make_lin(64)


def _final_body(z0_ref, z1_ref, disi_ref, b_ref, o_ref):
    o_ref[...] = ((z0_ref[...] + z1_ref[...]) * disi_ref[...]
                  + b_ref[...][None, :])


_final = pl.pallas_call(
    _final_body,
    grid=(GRID,),
    in_specs=[
        pl.BlockSpec((RB, 64), lambda i: (i, 0)),
        pl.BlockSpec((RB, 64), lambda i: (i, 0)),
        pl.BlockSpec((RB, 1), lambda i: (i, 0)),
        pl.BlockSpec((64,), lambda i: (0,)),
    ],
    out_specs=pl.BlockSpec((RB, 64), lambda i: (i, 0)),
    out_shape=jax.ShapeDtypeStruct((N, 64), jnp.float32),
)


def kernel(x, edge_index, W0, b0, W1, b1, W2, b2, g0, beta0, g1, beta1):
    src, dst = edge_index[0], edge_index[1]
    src3 = src.reshape(NW, NCHUNK, CHUNK)
    dst3 = dst.reshape(NW, NCHUNK, CHUNK)
    degp = _deg_kernel(src, dst)
    dis = _prep(degp)
    diso, disi = dis[:N, None], dis[N:, None]
    y0 = _lin0(x, diso, W0)
    z0 = _seg128(y0, src3, dst3)
    s0 = _stats(z0[0], z0[1], disi, b0)
    y1 = _lin1(z0[0], z0[1], disi, b0, s0, g0, beta0, diso, W1)
    z1 = _seg128(y1, src3, dst3)
    s1 = _stats(z1[0], z1[1], disi, b1)
    W2p = jnp.pad(W2, ((0, 0), (0, 24)))
    b2p = jnp.pad(b2, (0, 24))
    y2 = _lin2(z1[0], z1[1], disi, b1, s1, g1, beta1, diso, W2p)
    z2 = _seg64(y2, src3, dst3)
    out = _final(z2[0], z2[1], disi, b2p)
    return out[:, :40]


# speedup vs baseline: 12.3315x; 1.0680x over previous
import functools

import jax
import jax.numpy as jnp
from jax import lax
from jax.experimental import pallas as pl
from jax.experimental.pallas import tpu as pltpu
from jax.experimental.pallas import tpu_sc as plsc

N = 10000
E = 320000
EPS = 1e-5

_info = plsc.get_sparse_core_info()
NC = _info.num_cores
NS = _info.num_subcores
NW = NC * NS
LANES = _info.num_lanes

EPT = E // NW
CHUNK = 40
NCHUNK = EPT // CHUNK
SLAB = 624
ZCH = 16

_MESH = dict(mesh=plsc.VectorSubcoreMesh(core_axis_name="c",
                                         subcore_axis_name="s"),
             compiler_params=pltpu.CompilerParams(needs_layout_passes=False,
                                                  use_tc_tiling_on_sc=False))


@functools.partial(
    pl.kernel,
    out_type=jax.ShapeDtypeStruct((NW, 2 * N), jnp.float32),
    scratch_types=[
        pltpu.VMEM((EPT,), jnp.int32),
        pltpu.VMEM((EPT,), jnp.int32),
        pltpu.VMEM((2 * N,), jnp.float32),
    ],
    **_MESH,
)
def _deg_kernel(src, dst, out, sbuf, dbuf, hist):
    wid = lax.axis_index("s") * NC + lax.axis_index("c")
    base = wid * EPT
    pltpu.sync_copy(src.at[pl.ds(base, EPT)], sbuf)
    pltpu.sync_copy(dst.at[pl.ds(base, EPT)], dbuf)
    zeros16 = jnp.zeros((LANES,), jnp.float32)

    def zbody(i, c):
        hist[pl.ds(i * LANES, LANES)] = zeros16
        return c

    lax.fori_loop(0, (2 * N) // LANES, zbody, 0)
    ones16 = jnp.ones((LANES,), jnp.float32)
    off = jnp.full((LANES,), N, jnp.int32)

    def body(i, c):
        si = sbuf[pl.ds(i * LANES, LANES)]
        plsc.addupdate_scatter(hist, [si], ones16)
        di = dbuf[pl.ds(i * LANES, LANES)]
        plsc.addupdate_scatter(hist, [di + off], ones16)
        return c

    lax.fori_loop(0, EPT // LANES, body, 0)
    pltpu.sync_copy(hist, out.at[wid])


def _make_seg(D):
    @functools.partial(
        pl.kernel,
        out_type=jax.ShapeDtypeStruct((NC, N, D), jnp.float32),
        scratch_types=[
            pltpu.VMEM((NCHUNK, CHUNK), jnp.int32),
            pltpu.VMEM((NCHUNK, CHUNK), jnp.int32),
            [pltpu.VMEM((CHUNK, D), jnp.float32)] * 4,
            pltpu.VMEM((ZCH, D), jnp.float32),
            pltpu.VMEM_SHARED((N, D), jnp.float32),
            [pltpu.SemaphoreType.DMA] * 4,
            [pltpu.SemaphoreType.DMA] * 4,
            pltpu.SemaphoreType.DMA,
            pltpu.SemaphoreType.DMA,
        ],
        **_MESH,
    )
    def _seg(y, src3, dst3, out, sidx, didx, rows, zbuf, acc,
             gs, ss, isem0, isem1):
        cid = lax.axis_index("c")
        sid = lax.axis_index("s")
        wid = sid * NC + cid
        icp0 = pltpu.async_copy(src3.at[wid], sidx, isem0)
        icp1 = pltpu.async_copy(dst3.at[wid], didx, isem1)
        zeros16 = jnp.zeros((LANES,), jnp.float32)
        dl = D // LANES

        def zb(i, c):
            zbuf[i // dl, pl.ds((i % dl) * LANES, LANES)] = zeros16
            return c

        lax.fori_loop(0, ZCH * dl, zb, 0)
        rbase = sid * SLAB

        def zacc(i, c):
            pltpu.sync_copy(zbuf, acc.at[pl.ds(rbase + i * ZCH, ZCH)])
            return c

        lax.fori_loop(0, SLAB // ZCH, zacc, 0)

        @pl.when(sid == NS - 1)
        def _():
            lax.fori_loop(SLAB // ZCH, (N - (NS - 1) * SLAB) // ZCH, zacc, 0)

        icp0.wait()
        icp1.wait()
        plsc.subcore_barrier()

        for j in range(3):
            pltpu.async_copy(y.at[sidx.at[j]], rows[j], gs[j])

        def chunk(k, c):
            for b in range(4):
                @pl.when(k % 4 == b)
                def _(b=b):
                    pltpu.make_async_copy(y.at[sidx.at[k]], rows[b],
                                          gs[b]).wait()
                    pltpu.async_copy(rows[b], acc.at[didx.at[k]], ss[b],
                                     add=True)
                    b2 = (b + 3) % 4

                    @pl.when(k >= 1)
                    def _():
                        pltpu.make_async_copy(rows[b2], acc.at[didx.at[k - 1]],
                                              ss[b2]).wait()

                    @pl.when(k + 3 < NCHUNK)
                    def _():
                        pltpu.async_copy(y.at[sidx.at[k + 3]], rows[b2],
                                         gs[b2])

            return c

        lax.fori_loop(0, NCHUNK, chunk, 0)
        _bl = (NCHUNK - 1) % 4
        pltpu.make_async_copy(rows[_bl], acc.at[didx.at[NCHUNK - 1]],
                              ss[_bl]).wait()
        plsc.subcore_barrier()

        @pl.when(sid != NS - 1)
        def _():
            pltpu.sync_copy(acc.at[pl.ds(rbase, SLAB)],
                            out.at[cid, pl.ds(rbase, SLAB)])

        @pl.when(sid == NS - 1)
        def _():
            pltpu.sync_copy(acc.at[pl.ds(rbase, N - (NS - 1) * SLAB)],
                            out.at[cid, pl.ds(rbase, N - (NS - 1) * SLAB)])

    return _seg


_seg128 = _make_seg(128)
_seg64 = _make_seg(64)


RB = 1000
GRID = N // RB


def _prep_body(degp_ref, dis_ref):
    deg = jnp.sum(degp_ref[...], axis=0)
    dis_ref[...] = jnp.where(deg > 0.0,
                             lax.rsqrt(jnp.maximum(deg, 1.0)), 0.0)


_prep = pl.pallas_call(
    _prep_body,
    out_shape=jax.ShapeDtypeStruct((2 * N,), jnp.float32),
)


def _lin0_body(x_ref, diso_ref, w_ref, y_ref):
    y_ref[...] = jnp.dot(x_ref[...] * diso_ref[...], w_ref[...],
                         preferred_element_type=jnp.float32)


_lin0 = pl.pallas_call(
    _lin0_body,
    grid=(GRID,),
    in_specs=[
        pl.BlockSpec((RB, 128), lambda i: (i, 0)),
        pl.BlockSpec((RB, 1), lambda i: (i, 0)),
        pl.BlockSpec((128, 128), lambda i: (0, 0)),
    ],
    out_specs=pl.BlockSpec((RB, 128), lambda i: (i, 0)),
    out_shape=jax.ShapeDtypeStruct((N, 128), jnp.float32),
)


def _stats_body(z0_ref, z1_ref, disi_ref, b_ref, s_ref):
    h = ((z0_ref[...] + z1_ref[...]) * disi_ref[...]
         + b_ref[...][None, :])
    blk = jnp.stack([jnp.sum(h, axis=0), jnp.sum(h * h, axis=0)])

    @pl.when(pl.program_id(0) == 0)
    def _():
        s_ref[...] = blk

    @pl.when(pl.program_id(0) != 0)
    def _():
        s_ref[...] = s_ref[...] + blk


_stats = pl.pallas_call(
    _stats_body,
    grid=(GRID,),
    in_specs=[
        pl.BlockSpec((RB, 128), lambda i: (i, 0)),
        pl.BlockSpec((RB, 128), lambda i: (i, 0)),
        pl.BlockSpec((RB, 1), lambda i: (i, 0)),
        pl.BlockSpec((128,), lambda i: (0,)),
    ],
    out_specs=pl.BlockSpec((2, 128), lambda i: (0, 0)),
    out_shape=jax.ShapeDtypeStruct((2, 128), jnp.float32),
)


def _lin_body(z0_ref, z1_ref, disi_ref, b_ref, s_ref, g_ref, beta_ref,
              diso_ref, w_ref, y_ref):
    h = ((z0_ref[...] + z1_ref[...]) * disi_ref[...]
         + b_ref[...][None, :])
    mu = s_ref[0, :] / N
    var = jnp.maximum(s_ref[1, :] / N - mu * mu, 0.0)
    a = (h - mu[None, :]) * lax.rsqrt(var + EPS)[None, :]
    a = jnp.maximum(a * g_ref[...][None, :] + beta_ref[...][None, :], 0.0)
    y_ref[...] = jnp.dot(a * diso_ref[...], w_ref[...],
                         preferred_element_type=jnp.float32)


def _make_lin(dout):
    return pl.pallas_call(
        _lin_body,
        grid=(GRID,),
        in_specs=[
            pl.BlockSpec((RB, 128), lambda i: (i, 0)),
            pl.BlockSpec((RB, 128), lambda i: (i, 0)),
            pl.BlockSpec((RB, 1), lambda i: (i, 0)),
            pl.BlockSpec((128,), lambda i: (0,)),
            pl.BlockSpec((2, 128), lambda i: (0, 0)),
            pl.BlockSpec((128,), lambda i: (0,)),
            pl.BlockSpec((128,), lambda i: (0,)),
            pl.BlockSpec((RB, 1), lambda i: (i, 0)),
            pl.BlockSpec((128, dout), lambda i: (0, 0)),
        ],
        out_specs=pl.BlockSpec((RB, dout), lambda i: (i, 0)),
        out_shape=jax.ShapeDtypeStruct((N, dout), jnp.float32),
    )


_lin1 = _make_lin(128)
_lin2 = _make_lin(64)


def _final_body(z0_ref, z1_ref, disi_ref, b_ref, o_ref):
    o_ref[...] = ((z0_ref[...] + z1_ref[...]) * disi_ref[...]
                  + b_ref[...][None, :])


_final = pl.pallas_call(
    _final_body,
    grid=(GRID,),
    in_specs=[
        pl.BlockSpec((RB, 64), lambda i: (i, 0)),
        pl.BlockSpec((RB, 64), lambda i: (i, 0)),
        pl.BlockSpec((RB, 1), lambda i: (i, 0)),
        pl.BlockSpec((64,), lambda i: (0,)),
    ],
    out_specs=pl.BlockSpec((RB, 64), lambda i: (i, 0)),
    out_shape=jax.ShapeDtypeStruct((N, 64), jnp.float32),
)


def kernel(x, edge_index, W0, b0, W1, b1, W2, b2, g0, beta0, g1, beta1):
    src, dst = edge_index[0], edge_index[1]
    src3 = src.reshape(NW, NCHUNK, CHUNK)
    dst3 = dst.reshape(NW, NCHUNK, CHUNK)
    degp = _deg_kernel(src, dst)
    dis = _prep(degp)
    diso, disi = dis[:N, None], dis[N:, None]
    y0 = _lin0(x, diso, W0)
    z0 = _seg128(y0, src3, dst3)
    s0 = _stats(z0[0], z0[1], disi, b0)
    y1 = _lin1(z0[0], z0[1], disi, b0, s0, g0, beta0, diso, W1)
    z1 = _seg128(y1, src3, dst3)
    s1 = _stats(z1[0], z1[1], disi, b1)
    W2p = jnp.pad(W2, ((0, 0), (0, 24)))
    b2p = jnp.pad(b2, (0, 24))
    y2 = _lin2(z1[0], z1[1], disi, b1, s1, g1, beta1, diso, W2p)
    z2 = _seg64(y2, src3, dst3)
    out = _final(z2[0], z2[1], disi, b2p)
    return out[:, :40]


# speedup vs baseline: 12.5738x; 1.0196x over previous
import functools

import jax
import jax.numpy as jnp
from jax import lax
from jax.experimental import pallas as pl
from jax.experimental.pallas import tpu as pltpu
from jax.experimental.pallas import tpu_sc as plsc

N = 10000
E = 320000
EPS = 1e-5

_info = plsc.get_sparse_core_info()
NC = _info.num_cores
NS = _info.num_subcores
NW = NC * NS
LANES = _info.num_lanes

EPT = E // NW
CHUNK = 40
NCHUNK = EPT // CHUNK
SLAB = 624
ZCH = 16

_MESH = dict(mesh=plsc.VectorSubcoreMesh(core_axis_name="c",
                                         subcore_axis_name="s"),
             compiler_params=pltpu.CompilerParams(needs_layout_passes=False,
                                                  use_tc_tiling_on_sc=False))


NP = 10240
DSL = NP // NS
EPS_T = E // NS


@functools.partial(
    pl.kernel,
    out_type=jax.ShapeDtypeStruct((2 * N,), jnp.float32),
    scratch_types=[
        pltpu.VMEM((EPS_T,), jnp.int32),
        pltpu.VMEM((NP,), jnp.float32),
        pltpu.VMEM((NS, DSL), jnp.float32),
        pltpu.VMEM((DSL,), jnp.float32),
        pltpu.VMEM_SHARED((NS, NP), jnp.float32),
        pltpu.SemaphoreType.DMA,
        pltpu.SemaphoreType.DMA,
    ],
    **_MESH,
)
def _deg_kernel(src, dst, out, ibuf, hist, pbuf, rbuf, shist, isem, rsem):
    cid = lax.axis_index("c")
    sid = lax.axis_index("s")

    @pl.when(cid == 0)
    def _():
        pltpu.async_copy(src.at[pl.ds(sid * EPS_T, EPS_T)], ibuf, isem)

    @pl.when(cid != 0)
    def _():
        pltpu.async_copy(dst.at[pl.ds(sid * EPS_T, EPS_T)], ibuf, isem)

    zeros16 = jnp.zeros((LANES,), jnp.float32)

    def zbody(i, c):
        hist[pl.ds(i * LANES, LANES)] = zeros16
        return c

    lax.fori_loop(0, NP // LANES, zbody, 0)
    pltpu.make_async_copy(src.at[pl.ds(0, EPS_T)], ibuf, isem).wait()
    ones16 = jnp.ones((LANES,), jnp.float32)

    def body(i, c):
        plsc.addupdate_scatter(hist, [ibuf[pl.ds(i * LANES, LANES)]],
                               ones16)
        return c

    lax.fori_loop(0, EPS_T // LANES, body, 0)
    pltpu.sync_copy(hist, shist.at[sid])
    plsc.subcore_barrier()

    sbase = sid * DSL
    cps = [pltpu.async_copy(shist.at[j, pl.ds(sbase, DSL)], pbuf.at[j],
                            rsem) for j in range(NS)]
    for cp in cps:
        cp.wait()

    half3 = jnp.full((LANES,), 1.5, jnp.float32)
    magic = jnp.full((LANES,), 0x5f3759df, jnp.int32)

    def red(i, c):
        sl = pl.ds(i * LANES, LANES)
        v = pbuf[0, sl]
        for j in range(1, NS):
            v = v + pbuf[j, sl]
        m = v > 0.0
        xc = jnp.maximum(v, 1.0)
        half = xc * 0.5
        y = plsc.bitcast(magic - lax.shift_right_logical(
            plsc.bitcast(xc, jnp.int32), 1), jnp.float32)
        y = y * (half3 - half * y * y)
        y = y * (half3 - half * y * y)
        y = y * (half3 - half * y * y)
        rbuf[sl] = jnp.where(m, y, 0.0)
        return c

    lax.fori_loop(0, DSL // LANES, red, 0)

    @pl.when(sbase + DSL <= N)
    def _():
        pltpu.sync_copy(rbuf, out.at[pl.ds(cid * N + sbase, DSL)])

    @pl.when(jnp.logical_and(sbase < N, sbase + DSL > N))
    def _():
        pltpu.sync_copy(rbuf.at[pl.ds(0, N - (NS - 1) * DSL)],
                        out.at[pl.ds(cid * N + sbase,
                                     N - (NS - 1) * DSL)])


def _make_seg(D):
    @functools.partial(
        pl.kernel,
        out_type=jax.ShapeDtypeStruct((NC, N, D), jnp.float32),
        scratch_types=[
            pltpu.VMEM((NCHUNK, CHUNK), jnp.int32),
            pltpu.VMEM((NCHUNK, CHUNK), jnp.int32),
            [pltpu.VMEM((CHUNK, D), jnp.float32)] * 4,
            pltpu.VMEM((ZCH, D), jnp.float32),
            pltpu.VMEM_SHARED((N, D), jnp.float32),
            [pltpu.SemaphoreType.DMA] * 4,
            [pltpu.SemaphoreType.DMA] * 4,
            pltpu.SemaphoreType.DMA,
            pltpu.SemaphoreType.DMA,
        ],
        **_MESH,
    )
    def _seg(y, src3, dst3, out, sidx, didx, rows, zbuf, acc,
             gs, ss, isem0, isem1):
        cid = lax.axis_index("c")
        sid = lax.axis_index("s")
        wid = sid * NC + cid
        icp0 = pltpu.async_copy(src3.at[wid], sidx, isem0)
        icp1 = pltpu.async_copy(dst3.at[wid], didx, isem1)
        zeros16 = jnp.zeros((LANES,), jnp.float32)
        dl = D // LANES

        def zb(i, c):
            zbuf[i // dl, pl.ds((i % dl) * LANES, LANES)] = zeros16
            return c

        lax.fori_loop(0, ZCH * dl, zb, 0)
        rbase = sid * SLAB

        def zacc(i, c):
            pltpu.sync_copy(zbuf, acc.at[pl.ds(rbase + i * ZCH, ZCH)])
            return c

        lax.fori_loop(0, SLAB // ZCH, zacc, 0)

        @pl.when(sid == NS - 1)
        def _():
            lax.fori_loop(SLAB // ZCH, (N - (NS - 1) * SLAB) // ZCH, zacc, 0)

        icp0.wait()
        icp1.wait()
        plsc.subcore_barrier()

        for j in range(3):
            pltpu.async_copy(y.at[sidx.at[j]], rows[j], gs[j])

        def chunk(k, c):
            for b in range(4):
                @pl.when(k % 4 == b)
                def _(b=b):
                    pltpu.make_async_copy(y.at[sidx.at[k]], rows[b],
                                          gs[b]).wait()
                    pltpu.async_copy(rows[b], acc.at[didx.at[k]], ss[b],
                                     add=True)
                    b2 = (b + 3) % 4

                    @pl.when(k >= 1)
                    def _():
                        pltpu.make_async_copy(rows[b2], acc.at[didx.at[k - 1]],
                                              ss[b2]).wait()

                    @pl.when(k + 3 < NCHUNK)
                    def _():
                        pltpu.async_copy(y.at[sidx.at[k + 3]], rows[b2],
                                         gs[b2])

            return c

        lax.fori_loop(0, NCHUNK, chunk, 0)
        _bl = (NCHUNK - 1) % 4
        pltpu.make_async_copy(rows[_bl], acc.at[didx.at[NCHUNK - 1]],
                              ss[_bl]).wait()
        plsc.subcore_barrier()

        @pl.when(sid != NS - 1)
        def _():
            pltpu.sync_copy(acc.at[pl.ds(rbase, SLAB)],
                            out.at[cid, pl.ds(rbase, SLAB)])

        @pl.when(sid == NS - 1)
        def _():
            pltpu.sync_copy(acc.at[pl.ds(rbase, N - (NS - 1) * SLAB)],
                            out.at[cid, pl.ds(rbase, N - (NS - 1) * SLAB)])

    return _seg


_seg128 = _make_seg(128)
_seg64 = _make_seg(64)


RB = 1000
GRID = N // RB


def _lin0_body(x_ref, diso_ref, w_ref, y_ref):
    y_ref[...] = jnp.dot(x_ref[...] * diso_ref[...], w_ref[...],
                         preferred_element_type=jnp.float32)


_lin0 = pl.pallas_call(
    _lin0_body,
    grid=(GRID,),
    in_specs=[
        pl.BlockSpec((RB, 128), lambda i: (i, 0)),
        pl.BlockSpec((RB, 1), lambda i: (i, 0)),
        pl.BlockSpec((128, 128), lambda i: (0, 0)),
    ],
    out_specs=pl.BlockSpec((RB, 128), lambda i: (i, 0)),
    out_shape=jax.ShapeDtypeStruct((N, 128), jnp.float32),
)


def _linstats_body(z0_ref, z1_ref, disi_ref, b_ref, g_ref, beta_ref,
                   diso_ref, w_ref, y_ref, sscr):
    p = pl.program_id(0)
    i = pl.program_id(1)
    h = ((z0_ref[...] + z1_ref[...]) * disi_ref[...]
         + b_ref[...][None, :])

    @pl.when(p == 0)
    def _():
        blk = jnp.stack([jnp.sum(h, axis=0), jnp.sum(h * h, axis=0)])

        @pl.when(i == 0)
        def _():
            sscr[...] = blk

        @pl.when(i > 0)
        def _():
            sscr[...] = sscr[...] + blk

    @pl.when(p == 1)
    def _():
        s = sscr[...]
        mu = s[0, :] / N
        var = jnp.maximum(s[1, :] / N - mu * mu, 0.0)
        a = (h - mu[None, :]) * lax.rsqrt(var + EPS)[None, :]
        a = jnp.maximum(a * g_ref[...][None, :] + beta_ref[...][None, :],
                        0.0)
        y_ref[...] = jnp.dot(a * diso_ref[...], w_ref[...],
                             preferred_element_type=jnp.float32)


def _make_linstats(dout):
    return pl.pallas_call(
        _linstats_body,
        grid=(2, GRID),
        in_specs=[
            pl.BlockSpec((RB, 128), lambda p, i: (i, 0)),
            pl.BlockSpec((RB, 128), lambda p, i: (i, 0)),
            pl.BlockSpec((RB, 1), lambda p, i: (i, 0)),
            pl.BlockSpec((128,), lambda p, i: (0,)),
            pl.BlockSpec((128,), lambda p, i: (0,)),
            pl.BlockSpec((128,), lambda p, i: (0,)),
            pl.BlockSpec((RB, 1), lambda p, i: (i, 0)),
            pl.BlockSpec((128, dout), lambda p, i: (0, 0)),
        ],
        out_specs=pl.BlockSpec((RB, dout), lambda p, i: (i, 0)),
        out_shape=jax.ShapeDtypeStruct((N, dout), jnp.float32),
        scratch_shapes=[pltpu.VMEM((2, 128), jnp.float32)],
    )


_linstats1 = _make_linstats(128)
_linstats2 = _make_linstats(64)


def _final_body(z0_ref, z1_ref, disi_ref, b_ref, o_ref):
    o_ref[...] = ((z0_ref[...] + z1_ref[...]) * disi_ref[...]
                  + b_ref[...][None, :])


_final = pl.pallas_call(
    _final_body,
    grid=(GRID,),
    in_specs=[
        pl.BlockSpec((RB, 64), lambda i: (i, 0)),
        pl.BlockSpec((RB, 64), lambda i: (i, 0)),
        pl.BlockSpec((RB, 1), lambda i: (i, 0)),
        pl.BlockSpec((64,), lambda i: (0,)),
    ],
    out_specs=pl.BlockSpec((RB, 64), lambda i: (i, 0)),
    out_shape=jax.ShapeDtypeStruct((N, 64), jnp.float32),
)


def kernel(x, edge_index, W0, b0, W1, b1, W2, b2, g0, beta0, g1, beta1):
    src, dst = edge_index[0], edge_index[1]
    src3 = src.reshape(NW, NCHUNK, CHUNK)
    dst3 = dst.reshape(NW, NCHUNK, CHUNK)
    dis = _deg_kernel(src, dst)
    diso, disi = dis[:N, None], dis[N:, None]
    y0 = _lin0(x, diso, W0)
    z0 = _seg128(y0, src3, dst3)
    y1 = _linstats1(z0[0], z0[1], disi, b0, g0, beta0, diso, W1)
    z1 = _seg128(y1, src3, dst3)
    W2p = jnp.pad(W2, ((0, 0), (0, 24)))
    b2p = jnp.pad(b2, (0, 24))
    y2 = _linstats2(z1[0], z1[1], disi, b1, g1, beta1, diso, W2p)
    z2 = _seg64(y2, src3, dst3)
    out = _final(z2[0], z2[1], disi, b2p)
    return out[:, :40]


# speedup vs baseline: 13.2510x; 1.0539x over previous
import functools

import jax
import jax.numpy as jnp
from jax import lax
from jax.experimental import pallas as pl
from jax.experimental.pallas import tpu as pltpu
from jax.experimental.pallas import tpu_sc as plsc

N = 10000
E = 320000
EPS = 1e-5

_info = plsc.get_sparse_core_info()
NC = _info.num_cores
NS = _info.num_subcores
NW = NC * NS
LANES = _info.num_lanes

EPT = E // NW
CHUNK = 40
NCHUNK = EPT // CHUNK
SLAB = 624
ZCH = 16

_MESH = dict(mesh=plsc.VectorSubcoreMesh(core_axis_name="c",
                                         subcore_axis_name="s"),
             compiler_params=pltpu.CompilerParams(needs_layout_passes=False,
                                                  use_tc_tiling_on_sc=False))


NP = 10240
DSL = NP // NS
EPS_T = E // NS


@functools.partial(
    pl.kernel,
    out_type=jax.ShapeDtypeStruct((2 * N,), jnp.float32),
    scratch_types=[
        pltpu.VMEM((EPS_T,), jnp.int32),
        pltpu.VMEM((NP,), jnp.float32),
        pltpu.VMEM((NS, DSL), jnp.float32),
        pltpu.VMEM((DSL,), jnp.float32),
        pltpu.VMEM_SHARED((NS, NP), jnp.float32),
        pltpu.SemaphoreType.DMA,
        pltpu.SemaphoreType.DMA,
    ],
    **_MESH,
)
def _deg_kernel(src, dst, out, ibuf, hist, pbuf, rbuf, shist, isem, rsem):
    cid = lax.axis_index("c")
    sid = lax.axis_index("s")

    @pl.when(cid == 0)
    def _():
        pltpu.async_copy(src.at[pl.ds(sid * EPS_T, EPS_T)], ibuf, isem)

    @pl.when(cid != 0)
    def _():
        pltpu.async_copy(dst.at[pl.ds(sid * EPS_T, EPS_T)], ibuf, isem)

    zeros16 = jnp.zeros((LANES,), jnp.float32)

    def zbody(i, c):
        hist[pl.ds(i * LANES, LANES)] = zeros16
        return c

    lax.fori_loop(0, NP // LANES, zbody, 0)
    pltpu.make_async_copy(src.at[pl.ds(0, EPS_T)], ibuf, isem).wait()
    ones16 = jnp.ones((LANES,), jnp.float32)

    def body(i, c):
        plsc.addupdate_scatter(hist, [ibuf[pl.ds(i * LANES, LANES)]],
                               ones16)
        return c

    lax.fori_loop(0, EPS_T // LANES, body, 0)
    pltpu.sync_copy(hist, shist.at[sid])
    plsc.subcore_barrier()

    sbase = sid * DSL
    cps = [pltpu.async_copy(shist.at[j, pl.ds(sbase, DSL)], pbuf.at[j],
                            rsem) for j in range(NS)]
    for cp in cps:
        cp.wait()

    half3 = jnp.full((LANES,), 1.5, jnp.float32)
    magic = jnp.full((LANES,), 0x5f3759df, jnp.int32)

    def red(i, c):
        sl = pl.ds(i * LANES, LANES)
        v = pbuf[0, sl]
        for j in range(1, NS):
            v = v + pbuf[j, sl]
        m = v > 0.0
        xc = jnp.maximum(v, 1.0)
        half = xc * 0.5
        y = plsc.bitcast(magic - lax.shift_right_logical(
            plsc.bitcast(xc, jnp.int32), 1), jnp.float32)
        y = y * (half3 - half * y * y)
        y = y * (half3 - half * y * y)
        y = y * (half3 - half * y * y)
        rbuf[sl] = jnp.where(m, y, 0.0)
        return c

    lax.fori_loop(0, DSL // LANES, red, 0)

    @pl.when(sbase + DSL <= N)
    def _():
        pltpu.sync_copy(rbuf, out.at[pl.ds(cid * N + sbase, DSL)])

    @pl.when(jnp.logical_and(sbase < N, sbase + DSL > N))
    def _():
        pltpu.sync_copy(rbuf.at[pl.ds(0, N - (NS - 1) * DSL)],
                        out.at[pl.ds(cid * N + sbase,
                                     N - (NS - 1) * DSL)])


def _make_seg(D):
    @functools.partial(
        pl.kernel,
        out_type=jax.ShapeDtypeStruct((NC, N, D), jnp.float32),
        scratch_types=[
            pltpu.VMEM((NCHUNK, CHUNK), jnp.int32),
            pltpu.VMEM((NCHUNK, CHUNK), jnp.int32),
            [pltpu.VMEM((CHUNK, D), jnp.float32)] * 4,
            pltpu.VMEM((ZCH, D), jnp.float32),
            pltpu.VMEM_SHARED((N, D), jnp.float32),
            [pltpu.SemaphoreType.DMA] * 4,
            [pltpu.SemaphoreType.DMA] * 4,
            pltpu.SemaphoreType.DMA,
            pltpu.SemaphoreType.DMA,
        ],
        **_MESH,
    )
    def _seg(y, src3, dst3, out, sidx, didx, rows, zbuf, acc,
             gs, ss, isem0, isem1):
        cid = lax.axis_index("c")
        sid = lax.axis_index("s")
        wid = sid * NC + cid
        icp0 = pltpu.async_copy(src3.at[wid], sidx, isem0)
        icp1 = pltpu.async_copy(dst3.at[wid], didx, isem1)
        zeros16 = jnp.zeros((LANES,), jnp.float32)
        dl = D // LANES

        def zb(i, c):
            zbuf[i // dl, pl.ds((i % dl) * LANES, LANES)] = zeros16
            return c

        lax.fori_loop(0, ZCH * dl, zb, 0)
        rbase = sid * SLAB

        def zacc(i, c):
            pltpu.sync_copy(zbuf, acc.at[pl.ds(rbase + i * ZCH, ZCH)])
            return c

        lax.fori_loop(0, SLAB // ZCH, zacc, 0)

        @pl.when(sid == NS - 1)
        def _():
            lax.fori_loop(SLAB // ZCH, (N - (NS - 1) * SLAB) // ZCH, zacc, 0)

        icp0.wait()
        icp1.wait()
        plsc.subcore_barrier()

        for j in range(3):
            pltpu.async_copy(y.at[sidx.at[j]], rows[j], gs[j])

        def chunk(k, c):
            for b in range(4):
                @pl.when(k % 4 == b)
                def _(b=b):
                    pltpu.make_async_copy(y.at[sidx.at[k]], rows[b],
                                          gs[b]).wait()
                    pltpu.async_copy(rows[b], acc.at[didx.at[k]], ss[b],
                                     add=True)
                    b2 = (b + 3) % 4

                    @pl.when(k >= 1)
                    def _():
                        pltpu.make_async_copy(rows[b2], acc.at[didx.at[k - 1]],
                                              ss[b2]).wait()

                    @pl.when(k + 3 < NCHUNK)
                    def _():
                        pltpu.async_copy(y.at[sidx.at[k + 3]], rows[b2],
                                         gs[b2])

            return c

        lax.fori_loop(0, NCHUNK, chunk, 0)
        _bl = (NCHUNK - 1) % 4
        pltpu.make_async_copy(rows[_bl], acc.at[didx.at[NCHUNK - 1]],
                              ss[_bl]).wait()
        plsc.subcore_barrier()

        @pl.when(sid != NS - 1)
        def _():
            pltpu.sync_copy(acc.at[pl.ds(rbase, SLAB)],
                            out.at[cid, pl.ds(rbase, SLAB)])

        @pl.when(sid == NS - 1)
        def _():
            pltpu.sync_copy(acc.at[pl.ds(rbase, N - (NS - 1) * SLAB)],
                            out.at[cid, pl.ds(rbase, N - (NS - 1) * SLAB)])

    return _seg


_seg128 = _make_seg(128)
_seg64 = _make_seg(64)


RB = 1000
GRID = N // RB


def _lin0_body(x_ref, diso_ref, w_ref, y_ref):
    y_ref[...] = jnp.dot(x_ref[...] * diso_ref[...], w_ref[...],
                         preferred_element_type=jnp.float32)


_lin0 = pl.pallas_call(
    _lin0_body,
    grid=(GRID,),
    in_specs=[
        pl.BlockSpec((RB, 128), lambda i: (i, 0)),
        pl.BlockSpec((RB, 1), lambda i: (i, 0)),
        pl.BlockSpec((128, 128), lambda i: (0, 0)),
    ],
    out_specs=pl.BlockSpec((RB, 128), lambda i: (i, 0)),
    out_shape=jax.ShapeDtypeStruct((N, 128), jnp.float32),
)


def _linstats_body(z0_ref, z1_ref, disi_ref, b_ref, g_ref, beta_ref,
                   diso_ref, w_ref, y_ref, sscr, hscr):
    p = pl.program_id(0)
    i = pl.program_id(1)

    @pl.when(p == 0)
    def _():
        h = ((z0_ref[0] + z1_ref[0]) * disi_ref[...]
             + b_ref[...][None, :])
        hscr[i, :, :] = h
        blk = jnp.stack([jnp.sum(h, axis=0), jnp.sum(h * h, axis=0)])

        @pl.when(i == 0)
        def _():
            sscr[...] = blk

        @pl.when(i > 0)
        def _():
            sscr[...] = sscr[...] + blk

    @pl.when(p == 1)
    def _():
        h = hscr[i, :, :]
        s = sscr[...]
        mu = s[0, :] / N
        var = jnp.maximum(s[1, :] / N - mu * mu, 0.0)
        a = (h - mu[None, :]) * lax.rsqrt(var + EPS)[None, :]
        a = jnp.maximum(a * g_ref[...][None, :] + beta_ref[...][None, :],
                        0.0)
        y_ref[...] = jnp.dot(a * diso_ref[...], w_ref[...],
                             preferred_element_type=jnp.float32)


def _make_linstats(dout):
    return pl.pallas_call(
        _linstats_body,
        grid=(2, GRID),
        in_specs=[
            pl.BlockSpec((1, RB, 128), lambda p, i: (0, i, 0)),
            pl.BlockSpec((1, RB, 128), lambda p, i: (1, i, 0)),
            pl.BlockSpec((RB, 1), lambda p, i: (i, 0)),
            pl.BlockSpec((128,), lambda p, i: (0,)),
            pl.BlockSpec((128,), lambda p, i: (0,)),
            pl.BlockSpec((128,), lambda p, i: (0,)),
            pl.BlockSpec((RB, 1), lambda p, i: (i, 0)),
            pl.BlockSpec((128, dout), lambda p, i: (0, 0)),
        ],
        out_specs=pl.BlockSpec((RB, dout), lambda p, i: (i, 0)),
        out_shape=jax.ShapeDtypeStruct((N, dout), jnp.float32),
        scratch_shapes=[pltpu.VMEM((2, 128), jnp.float32),
                        pltpu.VMEM((GRID, RB, 128), jnp.float32)],
    )


_linstats1 = _make_linstats(128)
_linstats2 = _make_linstats(64)


def _final_body(z0_ref, z1_ref, disi_ref, b_ref, o_ref):
    o_ref[...] = ((z0_ref[0] + z1_ref[0]) * disi_ref[...]
                  + b_ref[...][None, :])[:, :40]


_final = pl.pallas_call(
    _final_body,
    grid=(GRID,),
    in_specs=[
        pl.BlockSpec((1, RB, 64), lambda i: (0, i, 0)),
        pl.BlockSpec((1, RB, 64), lambda i: (1, i, 0)),
        pl.BlockSpec((RB, 1), lambda i: (i, 0)),
        pl.BlockSpec((64,), lambda i: (0,)),
    ],
    out_specs=pl.BlockSpec((RB, 40), lambda i: (i, 0)),
    out_shape=jax.ShapeDtypeStruct((N, 40), jnp.float32),
)


def kernel(x, edge_index, W0, b0, W1, b1, W2, b2, g0, beta0, g1, beta1):
    src, dst = edge_index[0], edge_index[1]
    src3 = src.reshape(NW, NCHUNK, CHUNK)
    dst3 = dst.reshape(NW, NCHUNK, CHUNK)
    dis = _deg_kernel(src, dst)
    diso, disi = dis[:N, None], dis[N:, None]
    y0 = _lin0(x, diso, W0)
    z0 = _seg128(y0, src3, dst3)
    y1 = _linstats1(z0, z0, disi, b0, g0, beta0, diso, W1)
    z1 = _seg128(y1, src3, dst3)
    W2p = jnp.pad(W2, ((0, 0), (0, 24)))
    b2p = jnp.pad(b2, (0, 24))
    y2 = _linstats2(z1, z1, disi, b1, g1, beta1, diso, W2p)
    z2 = _seg64(y2, src3, dst3)
    return _final(z2, z2, disi, b2p)


# speedup vs baseline: 13.4869x; 1.0178x over previous
import functools

import jax
import jax.numpy as jnp
from jax import lax
from jax.experimental import pallas as pl
from jax.experimental.pallas import tpu as pltpu
from jax.experimental.pallas import tpu_sc as plsc

N = 10000
E = 320000
EPS = 1e-5

_info = plsc.get_sparse_core_info()
NC = _info.num_cores
NS = _info.num_subcores
NW = NC * NS
LANES = _info.num_lanes

EPT = E // NW
CHUNK = 40
NCHUNK = EPT // CHUNK
SLAB = 624
ZCH = 16

_MESH = dict(mesh=plsc.VectorSubcoreMesh(core_axis_name="c",
                                         subcore_axis_name="s"),
             compiler_params=pltpu.CompilerParams(needs_layout_passes=False,
                                                  use_tc_tiling_on_sc=False))


NP = 10240
DSL = NP // NS
EPS_T = E // NS


@functools.partial(
    pl.kernel,
    out_type=jax.ShapeDtypeStruct((2 * N,), jnp.float32),
    scratch_types=[
        pltpu.VMEM((EPS_T,), jnp.int32),
        pltpu.VMEM((NP,), jnp.float32),
        pltpu.VMEM((NS, DSL), jnp.float32),
        pltpu.VMEM((DSL,), jnp.float32),
        pltpu.VMEM_SHARED((NS, NP), jnp.float32),
        pltpu.SemaphoreType.DMA,
        pltpu.SemaphoreType.DMA,
    ],
    **_MESH,
)
def _deg_kernel(src, dst, out, ibuf, hist, pbuf, rbuf, shist, isem, rsem):
    cid = lax.axis_index("c")
    sid = lax.axis_index("s")

    @pl.when(cid == 0)
    def _():
        pltpu.async_copy(src.at[pl.ds(sid * EPS_T, EPS_T)], ibuf, isem)

    @pl.when(cid != 0)
    def _():
        pltpu.async_copy(dst.at[pl.ds(sid * EPS_T, EPS_T)], ibuf, isem)

    zeros16 = jnp.zeros((LANES,), jnp.float32)

    def zbody(i, c):
        hist[pl.ds(i * LANES, LANES)] = zeros16
        return c

    lax.fori_loop(0, NP // LANES, zbody, 0)
    pltpu.make_async_copy(src.at[pl.ds(0, EPS_T)], ibuf, isem).wait()
    ones16 = jnp.ones((LANES,), jnp.float32)

    def body(i, c):
        plsc.addupdate_scatter(hist, [ibuf[pl.ds(i * LANES, LANES)]],
                               ones16)
        return c

    lax.fori_loop(0, EPS_T // LANES, body, 0)
    pltpu.sync_copy(hist, shist.at[sid])
    plsc.subcore_barrier()

    sbase = sid * DSL
    cps = [pltpu.async_copy(shist.at[j, pl.ds(sbase, DSL)], pbuf.at[j],
                            rsem) for j in range(NS)]
    for cp in cps:
        cp.wait()

    half3 = jnp.full((LANES,), 1.5, jnp.float32)
    magic = jnp.full((LANES,), 0x5f3759df, jnp.int32)

    def red(i, c):
        sl = pl.ds(i * LANES, LANES)
        v = pbuf[0, sl]
        for j in range(1, NS):
            v = v + pbuf[j, sl]
        m = v > 0.0
        xc = jnp.maximum(v, 1.0)
        half = xc * 0.5
        y = plsc.bitcast(magic - lax.shift_right_logical(
            plsc.bitcast(xc, jnp.int32), 1), jnp.float32)
        y = y * (half3 - half * y * y)
        y = y * (half3 - half * y * y)
        y = y * (half3 - half * y * y)
        rbuf[sl] = jnp.where(m, y, 0.0)
        return c

    lax.fori_loop(0, DSL // LANES, red, 0)

    @pl.when(sbase + DSL <= N)
    def _():
        pltpu.sync_copy(rbuf, out.at[pl.ds(cid * N + sbase, DSL)])

    @pl.when(jnp.logical_and(sbase < N, sbase + DSL > N))
    def _():
        pltpu.sync_copy(rbuf.at[pl.ds(0, N - (NS - 1) * DSL)],
                        out.at[pl.ds(cid * N + sbase,
                                     N - (NS - 1) * DSL)])


def _make_seg(D):
    @functools.partial(
        pl.kernel,
        out_type=jax.ShapeDtypeStruct((NC, N, D), jnp.float32),
        scratch_types=[
            pltpu.VMEM((NCHUNK, CHUNK), jnp.int32),
            pltpu.VMEM((NCHUNK, CHUNK), jnp.int32),
            [pltpu.VMEM((CHUNK, D), jnp.float32)] * 4,
            pltpu.VMEM((ZCH, D), jnp.float32),
            pltpu.VMEM_SHARED((N, D), jnp.float32),
            [pltpu.SemaphoreType.DMA] * 4,
            [pltpu.SemaphoreType.DMA] * 4,
            pltpu.SemaphoreType.DMA,
            pltpu.SemaphoreType.DMA,
        ],
        **_MESH,
    )
    def _seg(y, src3, dst3, out, sidx, didx, rows, zbuf, acc,
             gs, ss, isem0, isem1):
        cid = lax.axis_index("c")
        sid = lax.axis_index("s")
        wid = sid * NC + cid
        icp0 = pltpu.async_copy(src3.at[wid], sidx, isem0)
        icp1 = pltpu.async_copy(dst3.at[wid], didx, isem1)
        zeros16 = jnp.zeros((LANES,), jnp.float32)
        dl = D // LANES

        def zb(i, c):
            zbuf[i // dl, pl.ds((i % dl) * LANES, LANES)] = zeros16
            return c

        lax.fori_loop(0, ZCH * dl, zb, 0)
        rbase = sid * SLAB

        def zacc(i, c):
            pltpu.sync_copy(zbuf, acc.at[pl.ds(rbase + i * ZCH, ZCH)])
            return c

        lax.fori_loop(0, SLAB // ZCH, zacc, 0)

        @pl.when(sid == NS - 1)
        def _():
            lax.fori_loop(SLAB // ZCH, (N - (NS - 1) * SLAB) // ZCH, zacc, 0)

        icp0.wait()
        icp1.wait()
        plsc.subcore_barrier()

        for j in range(3):
            pltpu.async_copy(y.at[sidx.at[j]], rows[j], gs[j])

        def chunk(k, c):
            for b in range(4):
                @pl.when(k % 4 == b)
                def _(b=b):
                    pltpu.make_async_copy(y.at[sidx.at[k]], rows[b],
                                          gs[b]).wait()
                    pltpu.async_copy(rows[b], acc.at[didx.at[k]], ss[b],
                                     add=True)
                    b2 = (b + 3) % 4

                    @pl.when(k >= 1)
                    def _():
                        pltpu.make_async_copy(rows[b2], acc.at[didx.at[k - 1]],
                                              ss[b2]).wait()

                    @pl.when(k + 3 < NCHUNK)
                    def _():
                        pltpu.async_copy(y.at[sidx.at[k + 3]], rows[b2],
                                         gs[b2])

            return c

        lax.fori_loop(0, NCHUNK, chunk, 0)
        _bl = (NCHUNK - 1) % 4
        pltpu.make_async_copy(rows[_bl], acc.at[didx.at[NCHUNK - 1]],
                              ss[_bl]).wait()
        plsc.subcore_barrier()

        @pl.when(sid != NS - 1)
        def _():
            pltpu.sync_copy(acc.at[pl.ds(rbase, SLAB)],
                            out.at[cid, pl.ds(rbase, SLAB)])

        @pl.when(sid == NS - 1)
        def _():
            pltpu.sync_copy(acc.at[pl.ds(rbase, N - (NS - 1) * SLAB)],
                            out.at[cid, pl.ds(rbase, N - (NS - 1) * SLAB)])

    return _seg


_seg128 = _make_seg(128)
_seg64 = _make_seg(64)


RB = 1000
GRID = N // RB


def _lin0_body(x_ref, diso_ref, w_ref, y_ref):
    y_ref[...] = jnp.dot(x_ref[...] * diso_ref[...], w_ref[...],
                         preferred_element_type=jnp.float32)


_lin0 = pl.pallas_call(
    _lin0_body,
    grid=(GRID,),
    in_specs=[
        pl.BlockSpec((RB, 128), lambda i: (i, 0)),
        pl.BlockSpec((RB, 1), lambda i: (i, 0)),
        pl.BlockSpec((128, 128), lambda i: (0, 0)),
    ],
    out_specs=pl.BlockSpec((RB, 128), lambda i: (i, 0)),
    out_shape=jax.ShapeDtypeStruct((N, 128), jnp.float32),
)


def _linstats_body(z0_ref, z1_ref, disi_ref, b_ref, g_ref, beta_ref,
                   diso_ref, w_ref, y_ref, sscr, hscr):
    p = pl.program_id(0)
    i = pl.program_id(1)

    @pl.when(p == 0)
    def _():
        h = ((z0_ref[0] + z1_ref[0]) * disi_ref[...]
             + b_ref[...][None, :])
        hscr[i, :, :] = h
        blk = jnp.stack([jnp.sum(h, axis=0), jnp.sum(h * h, axis=0)])

        @pl.when(i == 0)
        def _():
            sscr[...] = blk

        @pl.when(i > 0)
        def _():
            sscr[...] = sscr[...] + blk

    @pl.when(p == 1)
    def _():
        h = hscr[i, :, :]
        s = sscr[...]
        mu = s[0, :] / N
        var = jnp.maximum(s[1, :] / N - mu * mu, 0.0)
        a = (h - mu[None, :]) * lax.rsqrt(var + EPS)[None, :]
        a = jnp.maximum(a * g_ref[...][None, :] + beta_ref[...][None, :],
                        0.0)
        y_ref[...] = jnp.dot(a * diso_ref[...], w_ref[...],
                             preferred_element_type=jnp.float32)


def _make_linstats(dout):
    return pl.pallas_call(
        _linstats_body,
        grid=(2, GRID),
        in_specs=[
            pl.BlockSpec((1, RB, 128),
                         lambda p, i: (0, jnp.where(p == 0, i, 0), 0)),
            pl.BlockSpec((1, RB, 128),
                         lambda p, i: (1, jnp.where(p == 0, i, 0), 0)),
            pl.BlockSpec((RB, 1),
                         lambda p, i: (jnp.where(p == 0, i, 0), 0)),
            pl.BlockSpec((128,), lambda p, i: (0,)),
            pl.BlockSpec((128,), lambda p, i: (0,)),
            pl.BlockSpec((128,), lambda p, i: (0,)),
            pl.BlockSpec((RB, 1),
                         lambda p, i: (jnp.where(p == 1, i, 0), 0)),
            pl.BlockSpec((128, dout), lambda p, i: (0, 0)),
        ],
        out_specs=pl.BlockSpec((RB, dout), lambda p, i: (i, 0)),
        out_shape=jax.ShapeDtypeStruct((N, dout), jnp.float32),
        scratch_shapes=[pltpu.VMEM((2, 128), jnp.float32),
                        pltpu.VMEM((GRID, RB, 128), jnp.float32)],
    )


_linstats1 = _make_linstats(128)
_linstats2 = _make_linstats(64)


def _final_body(z0_ref, z1_ref, disi_ref, b_ref, o_ref):
    o_ref[...] = ((z0_ref[0] + z1_ref[0]) * disi_ref[...]
                  + b_ref[...][None, :])[:, :40]


_final = pl.pallas_call(
    _final_body,
    grid=(GRID,),
    in_specs=[
        pl.BlockSpec((1, RB, 64), lambda i: (0, i, 0)),
        pl.BlockSpec((1, RB, 64), lambda i: (1, i, 0)),
        pl.BlockSpec((RB, 1), lambda i: (i, 0)),
        pl.BlockSpec((64,), lambda i: (0,)),
    ],
    out_specs=pl.BlockSpec((RB, 40), lambda i: (i, 0)),
    out_shape=jax.ShapeDtypeStruct((N, 40), jnp.float32),
)


def kernel(x, edge_index, W0, b0, W1, b1, W2, b2, g0, beta0, g1, beta1):
    src, dst = edge_index[0], edge_index[1]
    src3 = src.reshape(NW, NCHUNK, CHUNK)
    dst3 = dst.reshape(NW, NCHUNK, CHUNK)
    dis = _deg_kernel(src, dst)
    diso, disi = dis[:N, None], dis[N:, None]
    y0 = _lin0(x, diso, W0)
    z0 = _seg128(y0, src3, dst3)
    y1 = _linstats1(z0, z0, disi, b0, g0, beta0, diso, W1)
    z1 = _seg128(y1, src3, dst3)
    W2p = jnp.pad(W2, ((0, 0), (0, 24)))
    b2p = jnp.pad(b2, (0, 24))
    y2 = _linstats2(z1, z1, disi, b1, g1, beta1, diso, W2p)
    z2 = _seg64(y2, src3, dst3)
    return _final(z2, z2, disi, b2p)


# speedup vs baseline: 14.4844x; 1.0740x over previous
import functools

import jax
import jax.numpy as jnp
from jax import lax
from jax.experimental import pallas as pl
from jax.experimental.pallas import tpu as pltpu
from jax.experimental.pallas import tpu_sc as plsc

N = 10000
E = 320000
EPS = 1e-5

_info = plsc.get_sparse_core_info()
NC = _info.num_cores
NS = _info.num_subcores
NW = NC * NS
LANES = _info.num_lanes

EPT = E // NW
CHUNK = 40
NCHUNK = EPT // CHUNK
SLAB = 624
ZCH = 16

_MESH = dict(mesh=plsc.VectorSubcoreMesh(core_axis_name="c",
                                         subcore_axis_name="s"),
             compiler_params=pltpu.CompilerParams(needs_layout_passes=False,
                                                  use_tc_tiling_on_sc=False))


NP = 10240
DSL = NP // NS
EPS_T = E // NS


@functools.partial(
    pl.kernel,
    out_type=jax.ShapeDtypeStruct((2 * N,), jnp.float32),
    scratch_types=[
        pltpu.VMEM((EPS_T,), jnp.int32),
        pltpu.VMEM((NP,), jnp.float32),
        pltpu.VMEM((NS, DSL), jnp.float32),
        pltpu.VMEM((DSL,), jnp.float32),
        pltpu.VMEM_SHARED((NS, NP), jnp.float32),
        pltpu.SemaphoreType.DMA,
        pltpu.SemaphoreType.DMA,
    ],
    **_MESH,
)
def _deg_kernel(src, dst, out, ibuf, hist, pbuf, rbuf, shist, isem, rsem):
    cid = lax.axis_index("c")
    sid = lax.axis_index("s")

    @pl.when(cid == 0)
    def _():
        pltpu.async_copy(src.at[pl.ds(sid * EPS_T, EPS_T)], ibuf, isem)

    @pl.when(cid != 0)
    def _():
        pltpu.async_copy(dst.at[pl.ds(sid * EPS_T, EPS_T)], ibuf, isem)

    zeros16 = jnp.zeros((LANES,), jnp.float32)

    def zbody(i, c):
        hist[pl.ds(i * LANES, LANES)] = zeros16
        return c

    lax.fori_loop(0, NP // LANES, zbody, 0)
    pltpu.make_async_copy(src.at[pl.ds(0, EPS_T)], ibuf, isem).wait()
    ones16 = jnp.ones((LANES,), jnp.float32)

    def body(i, c):
        plsc.addupdate_scatter(hist, [ibuf[pl.ds(i * LANES, LANES)]],
                               ones16)
        return c

    lax.fori_loop(0, EPS_T // LANES, body, 0)
    pltpu.sync_copy(hist, shist.at[sid])
    plsc.subcore_barrier()

    sbase = sid * DSL
    cps = [pltpu.async_copy(shist.at[j, pl.ds(sbase, DSL)], pbuf.at[j],
                            rsem) for j in range(NS)]
    for cp in cps:
        cp.wait()

    half3 = jnp.full((LANES,), 1.5, jnp.float32)
    magic = jnp.full((LANES,), 0x5f3759df, jnp.int32)

    def red(i, c):
        sl = pl.ds(i * LANES, LANES)
        v = pbuf[0, sl]
        for j in range(1, NS):
            v = v + pbuf[j, sl]
        m = v > 0.0
        xc = jnp.maximum(v, 1.0)
        half = xc * 0.5
        y = plsc.bitcast(magic - lax.shift_right_logical(
            plsc.bitcast(xc, jnp.int32), 1), jnp.float32)
        y = y * (half3 - half * y * y)
        y = y * (half3 - half * y * y)
        y = y * (half3 - half * y * y)
        rbuf[sl] = jnp.where(m, y, 0.0)
        return c

    lax.fori_loop(0, DSL // LANES, red, 0)

    @pl.when(sbase + DSL <= N)
    def _():
        pltpu.sync_copy(rbuf, out.at[pl.ds(cid * N + sbase, DSL)])

    @pl.when(jnp.logical_and(sbase < N, sbase + DSL > N))
    def _():
        pltpu.sync_copy(rbuf.at[pl.ds(0, N - (NS - 1) * DSL)],
                        out.at[pl.ds(cid * N + sbase,
                                     N - (NS - 1) * DSL)])


def _make_seg(D):
    @functools.partial(
        pl.kernel,
        out_type=jax.ShapeDtypeStruct((NC, N, D), jnp.float32),
        scratch_types=[
            pltpu.VMEM((NCHUNK, CHUNK), jnp.int32),
            pltpu.VMEM((NCHUNK, CHUNK), jnp.int32),
            [pltpu.VMEM((CHUNK, D), jnp.float32)] * 5,
            pltpu.VMEM((ZCH, D), jnp.float32),
            pltpu.VMEM_SHARED((N, D), jnp.float32),
            [pltpu.SemaphoreType.DMA] * 5,
            [pltpu.SemaphoreType.DMA] * 5,
            pltpu.SemaphoreType.DMA,
            pltpu.SemaphoreType.DMA,
        ],
        **_MESH,
    )
    def _seg(y, src3, dst3, out, sidx, didx, rows, zbuf, acc,
             gs, ss, isem0, isem1):
        cid = lax.axis_index("c")
        sid = lax.axis_index("s")
        wid = sid * NC + cid
        icp0 = pltpu.async_copy(src3.at[wid], sidx, isem0)
        icp1 = pltpu.async_copy(dst3.at[wid], didx, isem1)
        zeros16 = jnp.zeros((LANES,), jnp.float32)
        dl = D // LANES

        def zb(i, c):
            zbuf[i // dl, pl.ds((i % dl) * LANES, LANES)] = zeros16
            return c

        lax.fori_loop(0, ZCH * dl, zb, 0)
        rbase = sid * SLAB

        def zacc(i, c):
            pltpu.sync_copy(zbuf, acc.at[pl.ds(rbase + i * ZCH, ZCH)])
            return c

        lax.fori_loop(0, SLAB // ZCH, zacc, 0)

        @pl.when(sid == NS - 1)
        def _():
            lax.fori_loop(SLAB // ZCH, (N - (NS - 1) * SLAB) // ZCH, zacc, 0)

        icp0.wait()
        icp1.wait()
        plsc.subcore_barrier()

        for j in range(4):
            pltpu.async_copy(y.at[sidx.at[j]], rows[j], gs[j])

        def chunk(k, c):
            for b in range(5):
                @pl.when(k % 5 == b)
                def _(b=b):
                    pltpu.make_async_copy(y.at[sidx.at[k]], rows[b],
                                          gs[b]).wait()
                    pltpu.async_copy(rows[b], acc.at[didx.at[k]], ss[b],
                                     add=True)
                    b2 = (b + 4) % 5

                    @pl.when(k >= 1)
                    def _():
                        pltpu.make_async_copy(rows[b2], acc.at[didx.at[k - 1]],
                                              ss[b2]).wait()

                    @pl.when(k + 4 < NCHUNK)
                    def _():
                        pltpu.async_copy(y.at[sidx.at[k + 4]], rows[b2],
                                         gs[b2])

            return c

        lax.fori_loop(0, NCHUNK, chunk, 0)
        _bl = (NCHUNK - 1) % 5
        pltpu.make_async_copy(rows[_bl], acc.at[didx.at[NCHUNK - 1]],
                              ss[_bl]).wait()
        plsc.subcore_barrier()

        @pl.when(sid != NS - 1)
        def _():
            pltpu.sync_copy(acc.at[pl.ds(rbase, SLAB)],
                            out.at[cid, pl.ds(rbase, SLAB)])

        @pl.when(sid == NS - 1)
        def _():
            pltpu.sync_copy(acc.at[pl.ds(rbase, N - (NS - 1) * SLAB)],
                            out.at[cid, pl.ds(rbase, N - (NS - 1) * SLAB)])

    return _seg


_seg128 = _make_seg(128)
_seg64 = _make_seg(64)


RB = 1000
GRID = N // RB


def _lin0_body(x_ref, diso_ref, w_ref, y_ref):
    y_ref[...] = jnp.dot(x_ref[...] * diso_ref[...], w_ref[...],
                         preferred_element_type=jnp.float32)


_lin0 = pl.pallas_call(
    _lin0_body,
    grid=(GRID,),
    in_specs=[
        pl.BlockSpec((RB, 128), lambda i: (i, 0)),
        pl.BlockSpec((RB, 1), lambda i: (i, 0)),
        pl.BlockSpec((128, 128), lambda i: (0, 0)),
    ],
    out_specs=pl.BlockSpec((RB, 128), lambda i: (i, 0)),
    out_shape=jax.ShapeDtypeStruct((N, 128), jnp.float32),
)


def _linstats_body(z0_ref, z1_ref, disi_ref, b_ref, g_ref, beta_ref,
                   diso_ref, w_ref, y_ref, sscr, hscr):
    p = pl.program_id(0)
    i = pl.program_id(1)

    @pl.when(p == 0)
    def _():
        h = ((z0_ref[0] + z1_ref[0]) * disi_ref[...]
             + b_ref[...][None, :])
        hscr[i, :, :] = h
        blk = jnp.stack([jnp.sum(h, axis=0), jnp.sum(h * h, axis=0)])

        @pl.when(i == 0)
        def _():
            sscr[...] = blk

        @pl.when(i > 0)
        def _():
            sscr[...] = sscr[...] + blk

    @pl.when(p == 1)
    def _():
        h = hscr[i, :, :]
        s = sscr[...]
        mu = s[0, :] / N
        var = jnp.maximum(s[1, :] / N - mu * mu, 0.0)
        a = (h - mu[None, :]) * lax.rsqrt(var + EPS)[None, :]
        a = jnp.maximum(a * g_ref[...][None, :] + beta_ref[...][None, :],
                        0.0)
        y_ref[...] = jnp.dot(a * diso_ref[...], w_ref[...],
                             preferred_element_type=jnp.float32)


def _make_linstats(dout):
    return pl.pallas_call(
        _linstats_body,
        grid=(2, GRID),
        in_specs=[
            pl.BlockSpec((1, RB, 128),
                         lambda p, i: (0, jnp.where(p == 0, i, 0), 0)),
            pl.BlockSpec((1, RB, 128),
                         lambda p, i: (1, jnp.where(p == 0, i, 0), 0)),
            pl.BlockSpec((RB, 1),
                         lambda p, i: (jnp.where(p == 0, i, 0), 0)),
            pl.BlockSpec((128,), lambda p, i: (0,)),
            pl.BlockSpec((128,), lambda p, i: (0,)),
            pl.BlockSpec((128,), lambda p, i: (0,)),
            pl.BlockSpec((RB, 1),
                         lambda p, i: (jnp.where(p == 1, i, 0), 0)),
            pl.BlockSpec((128, dout), lambda p, i: (0, 0)),
        ],
        out_specs=pl.BlockSpec((RB, dout), lambda p, i: (i, 0)),
        out_shape=jax.ShapeDtypeStruct((N, dout), jnp.float32),
        scratch_shapes=[pltpu.VMEM((2, 128), jnp.float32),
                        pltpu.VMEM((GRID, RB, 128), jnp.float32)],
    )


_linstats1 = _make_linstats(128)
_linstats2 = _make_linstats(64)


def _final_body(z0_ref, z1_ref, disi_ref, b_ref, o_ref):
    o_ref[...] = ((z0_ref[0] + z1_ref[0]) * disi_ref[...]
                  + b_ref[...][None, :])[:, :40]


_final = pl.pallas_call(
    _final_body,
    grid=(GRID,),
    in_specs=[
        pl.BlockSpec((1, RB, 64), lambda i: (0, i, 0)),
        pl.BlockSpec((1, RB, 64), lambda i: (1, i, 0)),
        pl.BlockSpec((RB, 1), lambda i: (i, 0)),
        pl.BlockSpec((64,), lambda i: (0,)),
    ],
    out_specs=pl.BlockSpec((RB, 40), lambda i: (i, 0)),
    out_shape=jax.ShapeDtypeStruct((N, 40), jnp.float32),
)


def kernel(x, edge_index, W0, b0, W1, b1, W2, b2, g0, beta0, g1, beta1):
    src, dst = edge_index[0], edge_index[1]
    src3 = src.reshape(NW, NCHUNK, CHUNK)
    dst3 = dst.reshape(NW, NCHUNK, CHUNK)
    dis = _deg_kernel(src, dst)
    diso, disi = dis[:N, None], dis[N:, None]
    y0 = _lin0(x, diso, W0)
    z0 = _seg128(y0, src3, dst3)
    y1 = _linstats1(z0, z0, disi, b0, g0, beta0, diso, W1)
    z1 = _seg128(y1, src3, dst3)
    W2p = jnp.pad(W2, ((0, 0), (0, 24)))
    b2p = jnp.pad(b2, (0, 24))
    y2 = _linstats2(z1, z1, disi, b1, g1, beta1, diso, W2p)
    z2 = _seg64(y2, src3, dst3)
    return _final(z2, z2, disi, b2p)


# speedup vs baseline: 15.0272x; 1.0375x over previous
import functools

import jax
import jax.numpy as jnp
from jax import lax
from jax.experimental import pallas as pl
from jax.experimental.pallas import tpu as pltpu
from jax.experimental.pallas import tpu_sc as plsc

N = 10000
E = 320000
EPS = 1e-5

_info = plsc.get_sparse_core_info()
NC = _info.num_cores
NS = _info.num_subcores
NW = NC * NS
LANES = _info.num_lanes

EPT = E // NW
CHUNK = 40
NCHUNK = EPT // CHUNK
SLAB = 624
ZCH = 16

_MESH = dict(mesh=plsc.VectorSubcoreMesh(core_axis_name="c",
                                         subcore_axis_name="s"),
             compiler_params=pltpu.CompilerParams(needs_layout_passes=False,
                                                  use_tc_tiling_on_sc=False))


NP = 10240
DSL = NP // NS
EPS_T = E // NS


@functools.partial(
    pl.kernel,
    out_type=jax.ShapeDtypeStruct((2 * N,), jnp.float32),
    scratch_types=[
        pltpu.VMEM((EPS_T,), jnp.int32),
        pltpu.VMEM((NP,), jnp.float32),
        pltpu.VMEM((NS, DSL), jnp.float32),
        pltpu.VMEM((DSL,), jnp.float32),
        pltpu.VMEM_SHARED((NS, NP), jnp.float32),
        pltpu.SemaphoreType.DMA,
        pltpu.SemaphoreType.DMA,
    ],
    **_MESH,
)
def _deg_kernel(src, dst, out, ibuf, hist, pbuf, rbuf, shist, isem, rsem):
    cid = lax.axis_index("c")
    sid = lax.axis_index("s")

    @pl.when(cid == 0)
    def _():
        pltpu.async_copy(src.at[pl.ds(sid * EPS_T, EPS_T)], ibuf, isem)

    @pl.when(cid != 0)
    def _():
        pltpu.async_copy(dst.at[pl.ds(sid * EPS_T, EPS_T)], ibuf, isem)

    zeros16 = jnp.zeros((LANES,), jnp.float32)

    def zbody(i, c):
        hist[pl.ds(i * LANES, LANES)] = zeros16
        return c

    lax.fori_loop(0, NP // LANES, zbody, 0)
    pltpu.make_async_copy(src.at[pl.ds(0, EPS_T)], ibuf, isem).wait()
    ones16 = jnp.ones((LANES,), jnp.float32)

    def body(i, c):
        plsc.addupdate_scatter(hist, [ibuf[pl.ds(i * LANES, LANES)]],
                               ones16)
        return c

    lax.fori_loop(0, EPS_T // LANES, body, 0)
    pltpu.sync_copy(hist, shist.at[sid])
    plsc.subcore_barrier()

    sbase = sid * DSL
    cps = [pltpu.async_copy(shist.at[j, pl.ds(sbase, DSL)], pbuf.at[j],
                            rsem) for j in range(NS)]
    for cp in cps:
        cp.wait()

    half3 = jnp.full((LANES,), 1.5, jnp.float32)
    magic = jnp.full((LANES,), 0x5f3759df, jnp.int32)

    def red(i, c):
        sl = pl.ds(i * LANES, LANES)
        v = pbuf[0, sl]
        for j in range(1, NS):
            v = v + pbuf[j, sl]
        m = v > 0.0
        xc = jnp.maximum(v, 1.0)
        half = xc * 0.5
        y = plsc.bitcast(magic - lax.shift_right_logical(
            plsc.bitcast(xc, jnp.int32), 1), jnp.float32)
        y = y * (half3 - half * y * y)
        y = y * (half3 - half * y * y)
        y = y * (half3 - half * y * y)
        rbuf[sl] = jnp.where(m, y, 0.0)
        return c

    lax.fori_loop(0, DSL // LANES, red, 0)

    @pl.when(sbase + DSL <= N)
    def _():
        pltpu.sync_copy(rbuf, out.at[pl.ds(cid * N + sbase, DSL)])

    @pl.when(jnp.logical_and(sbase < N, sbase + DSL > N))
    def _():
        pltpu.sync_copy(rbuf.at[pl.ds(0, N - (NS - 1) * DSL)],
                        out.at[pl.ds(cid * N + sbase,
                                     N - (NS - 1) * DSL)])


def _make_seg(D):
    @functools.partial(
        pl.kernel,
        out_type=jax.ShapeDtypeStruct((NC, N, D), jnp.float32),
        scratch_types=[
            pltpu.VMEM((NCHUNK, CHUNK), jnp.int32),
            pltpu.VMEM((NCHUNK, CHUNK), jnp.int32),
            [pltpu.VMEM((CHUNK, D), jnp.float32)] * 6,
            pltpu.VMEM_SHARED((N, D), jnp.float32),
            [pltpu.SemaphoreType.DMA] * 6,
            [pltpu.SemaphoreType.DMA] * 6,
            pltpu.SemaphoreType.DMA,
            pltpu.SemaphoreType.DMA,
        ],
        **_MESH,
    )
    def _seg(y, src3, dst3, out, sidx, didx, rows, acc,
             gs, ss, isem0, isem1):
        cid = lax.axis_index("c")
        sid = lax.axis_index("s")
        wid = sid * NC + cid
        icp0 = pltpu.async_copy(src3.at[wid], sidx, isem0)
        icp1 = pltpu.async_copy(dst3.at[wid], didx, isem1)
        zeros16 = jnp.zeros((LANES,), jnp.float32)
        dl = D // LANES

        def zb(i, c):
            rows[5][i // dl, pl.ds((i % dl) * LANES, LANES)] = zeros16
            return c

        lax.fori_loop(0, CHUNK * dl, zb, 0)
        rbase = sid * SLAB

        def zacc(i, c):
            pltpu.sync_copy(rows[5].at[pl.ds(0, ZCH)],
                            acc.at[pl.ds(rbase + i * ZCH, ZCH)])
            return c

        lax.fori_loop(0, SLAB // ZCH, zacc, 0)

        @pl.when(sid == NS - 1)
        def _():
            lax.fori_loop(SLAB // ZCH, (N - (NS - 1) * SLAB) // ZCH, zacc, 0)

        icp0.wait()
        icp1.wait()
        plsc.subcore_barrier()

        for j in range(5):
            pltpu.async_copy(y.at[sidx.at[j]], rows[j], gs[j])

        def chunk(k, c):
            for b in range(6):
                @pl.when(k % 6 == b)
                def _(b=b):
                    pltpu.make_async_copy(y.at[sidx.at[k]], rows[b],
                                          gs[b]).wait()
                    pltpu.async_copy(rows[b], acc.at[didx.at[k]], ss[b],
                                     add=True)
                    b2 = (b + 5) % 6

                    @pl.when(k >= 1)
                    def _():
                        pltpu.make_async_copy(rows[b2], acc.at[didx.at[k - 1]],
                                              ss[b2]).wait()

                    @pl.when(k + 5 < NCHUNK)
                    def _():
                        pltpu.async_copy(y.at[sidx.at[k + 5]], rows[b2],
                                         gs[b2])

            return c

        lax.fori_loop(0, NCHUNK, chunk, 0)
        _bl = (NCHUNK - 1) % 6
        pltpu.make_async_copy(rows[_bl], acc.at[didx.at[NCHUNK - 1]],
                              ss[_bl]).wait()
        plsc.subcore_barrier()

        @pl.when(sid != NS - 1)
        def _():
            pltpu.sync_copy(acc.at[pl.ds(rbase, SLAB)],
                            out.at[cid, pl.ds(rbase, SLAB)])

        @pl.when(sid == NS - 1)
        def _():
            pltpu.sync_copy(acc.at[pl.ds(rbase, N - (NS - 1) * SLAB)],
                            out.at[cid, pl.ds(rbase, N - (NS - 1) * SLAB)])

    return _seg


_seg128 = _make_seg(128)
_seg64 = _make_seg(64)


RB = 1000
GRID = N // RB


def _lin0_body(x_ref, diso_ref, w_ref, y_ref):
    y_ref[...] = jnp.dot(x_ref[...] * diso_ref[...], w_ref[...],
                         preferred_element_type=jnp.float32)


_lin0 = pl.pallas_call(
    _lin0_body,
    grid=(GRID,),
    in_specs=[
        pl.BlockSpec((RB, 128), lambda i: (i, 0)),
        pl.BlockSpec((RB, 1), lambda i: (i, 0)),
        pl.BlockSpec((128, 128), lambda i: (0, 0)),
    ],
    out_specs=pl.BlockSpec((RB, 128), lambda i: (i, 0)),
    out_shape=jax.ShapeDtypeStruct((N, 128), jnp.float32),
)


def _linstats_body(z0_ref, z1_ref, disi_ref, b_ref, g_ref, beta_ref,
                   diso_ref, w_ref, y_ref, sscr, hscr):
    p = pl.program_id(0)
    i = pl.program_id(1)

    @pl.when(p == 0)
    def _():
        h = ((z0_ref[0] + z1_ref[0]) * disi_ref[...]
             + b_ref[...][None, :])
        hscr[i, :, :] = h
        blk = jnp.stack([jnp.sum(h, axis=0), jnp.sum(h * h, axis=0)])

        @pl.when(i == 0)
        def _():
            sscr[...] = blk

        @pl.when(i > 0)
        def _():
            sscr[...] = sscr[...] + blk

    @pl.when(p == 1)
    def _():
        h = hscr[i, :, :]
        s = sscr[...]
        mu = s[0, :] / N
        var = jnp.maximum(s[1, :] / N - mu * mu, 0.0)
        a = (h - mu[None, :]) * lax.rsqrt(var + EPS)[None, :]
        a = jnp.maximum(a * g_ref[...][None, :] + beta_ref[...][None, :],
                        0.0)
        y_ref[...] = jnp.dot(a * diso_ref[...], w_ref[...],
                             preferred_element_type=jnp.float32)


def _make_linstats(dout):
    return pl.pallas_call(
        _linstats_body,
        grid=(2, GRID),
        in_specs=[
            pl.BlockSpec((1, RB, 128),
                         lambda p, i: (0, jnp.where(p == 0, i, 0), 0)),
            pl.BlockSpec((1, RB, 128),
                         lambda p, i: (1, jnp.where(p == 0, i, 0), 0)),
            pl.BlockSpec((RB, 1),
                         lambda p, i: (jnp.where(p == 0, i, 0), 0)),
            pl.BlockSpec((128,), lambda p, i: (0,)),
            pl.BlockSpec((128,), lambda p, i: (0,)),
            pl.BlockSpec((128,), lambda p, i: (0,)),
            pl.BlockSpec((RB, 1),
                         lambda p, i: (jnp.where(p == 1, i, 0), 0)),
            pl.BlockSpec((128, dout), lambda p, i: (0, 0)),
        ],
        out_specs=pl.BlockSpec((RB, dout), lambda p, i: (i, 0)),
        out_shape=jax.ShapeDtypeStruct((N, dout), jnp.float32),
        scratch_shapes=[pltpu.VMEM((2, 128), jnp.float32),
                        pltpu.VMEM((GRID, RB, 128), jnp.float32)],
    )


_linstats1 = _make_linstats(128)
_linstats2 = _make_linstats(64)


def _final_body(z0_ref, z1_ref, disi_ref, b_ref, o_ref):
    o_ref[...] = ((z0_ref[0] + z1_ref[0]) * disi_ref[...]
                  + b_ref[...][None, :])[:, :40]


_final = pl.pallas_call(
    _final_body,
    grid=(GRID,),
    in_specs=[
        pl.BlockSpec((1, RB, 64), lambda i: (0, i, 0)),
        pl.BlockSpec((1, RB, 64), lambda i: (1, i, 0)),
        pl.BlockSpec((RB, 1), lambda i: (i, 0)),
        pl.BlockSpec((64,), lambda i: (0,)),
    ],
    out_specs=pl.BlockSpec((RB, 40), lambda i: (i, 0)),
    out_shape=jax.ShapeDtypeStruct((N, 40), jnp.float32),
)


def kernel(x, edge_index, W0, b0, W1, b1, W2, b2, g0, beta0, g1, beta1):
    src, dst = edge_index[0], edge_index[1]
    src3 = src.reshape(NW, NCHUNK, CHUNK)
    dst3 = dst.reshape(NW, NCHUNK, CHUNK)
    dis = _deg_kernel(src, dst)
    diso, disi = dis[:N, None], dis[N:, None]
    y0 = _lin0(x, diso, W0)
    z0 = _seg128(y0, src3, dst3)
    y1 = _linstats1(z0, z0, disi, b0, g0, beta0, diso, W1)
    z1 = _seg128(y1, src3, dst3)
    W2p = jnp.pad(W2, ((0, 0), (0, 24)))
    b2p = jnp.pad(b2, (0, 24))
    y2 = _linstats2(z1, z1, disi, b1, g1, beta1, diso, W2p)
    z2 = _seg64(y2, src3, dst3)
    return _final(z2, z2, disi, b2p)
